# baseline (speedup 1.0000x reference)
import functools

import jax
import jax.numpy as jnp
from jax import lax
from jax.experimental import pallas as pl
from jax.experimental.pallas import tpu as pltpu
from jax.experimental.pallas import tpu_sc as plsc

B, L = 32, 20
V, E, H = 100000, 128, 512
M = B * (L + 1)


_NW = 32
_BP = 768
_B_PER_W = _BP // _NW


def _sc_gather_body(table_hbm, idx_hbm, out_hbm, idx_v, rows_v, sem):
    wid = lax.axis_index("s") * 2 + lax.axis_index("c")
    base = wid * _B_PER_W
    pltpu.sync_copy(idx_hbm.at[pl.ds(base, _B_PER_W)], idx_v)
    pltpu.async_copy(table_hbm.at[idx_v], rows_v, sem).wait()
    pltpu.sync_copy(rows_v, out_hbm.at[pl.ds(base, _B_PER_W)])


@functools.partial(jax.jit, static_argnums=())
def _sc_gather(table, idx_padded):
    k = pl.kernel(
        _sc_gather_body,
        out_type=jax.ShapeDtypeStruct((_BP, E), jnp.float32),
        mesh=plsc.VectorSubcoreMesh(core_axis_name="c", subcore_axis_name="s"),
        scratch_types=[
            pltpu.VMEM((_B_PER_W,), jnp.int32),
            pltpu.VMEM((_B_PER_W, E), jnp.float32),
            pltpu.SemaphoreType.DMA,
        ],
    )
    return k(table, idx_padded)


TV = 2048


def _dense_body(x_ref, w1_ref, b1_ref, w2_ref, b2_ref, out_ref, h_ref):
    @pl.when(pl.program_id(0) == 0)
    def _():
        h = jnp.dot(x_ref[...], w1_ref[...],
                    preferred_element_type=jnp.float32,
                    precision=lax.Precision.HIGHEST)
        h_ref[...] = (h + b1_ref[...]).astype(jnp.bfloat16)

    acc = jnp.dot(h_ref[...], w2_ref[...].astype(jnp.bfloat16),
                  preferred_element_type=jnp.float32)
    out_ref[...] = acc + b2_ref[...]


def _dense(x, W1, b1, W2, b2):
    nt = pl.cdiv(V, TV)
    return pl.pallas_call(
        _dense_body,
        grid=(nt,),
        in_specs=[
            pl.BlockSpec((M, E), lambda j: (0, 0)),
            pl.BlockSpec((E, H), lambda j: (0, 0)),
            pl.BlockSpec((1, H), lambda j: (0, 0)),
            pl.BlockSpec((H, TV), lambda j: (0, j)),
            pl.BlockSpec((1, TV), lambda j: (0, j)),
        ],
        out_specs=pl.BlockSpec((M, TV), lambda j: (0, j)),
        out_shape=jax.ShapeDtypeStruct((M, V), jnp.float32),
        scratch_shapes=[pltpu.VMEM((M, H), jnp.bfloat16)],
        compiler_params=pltpu.CompilerParams(
            dimension_semantics=("arbitrary",),
        ),
    )(x, W1, b1, W2, b2)


def kernel(features, captions, table, W1, b1, W2, b2):
    idx = captions.reshape(-1)
    idx_padded = jnp.concatenate(
        [idx, jnp.zeros((_BP - B * L,), jnp.int32)])
    emb = _sc_gather(table, idx_padded)[: B * L].reshape(B, L, E)
    x = jnp.concatenate([features[:, None, :], emb], axis=1).reshape(M, E)
    out = _dense(x, W1, b1.reshape(1, H), W2, b2.reshape(1, V))
    return out.reshape(B, L + 1, V)


# speedup vs baseline: 1.4620x; 1.4620x over previous
import jax
import jax.numpy as jnp
from jax import lax
from jax.experimental import pallas as pl
from jax.experimental.pallas import tpu as pltpu
from jax.experimental.pallas import tpu_sc as plsc

B, L = 32, 20
V, E, H = 100000, 128, 512
T = L + 1
TP = 24
MP = B * TP


_NW = 32
_BP = 768
_B_PER_W = _BP // _NW


def _sc_gather_body(table_hbm, idx_hbm, out_hbm, idx_v, rows_v, sem):
    wid = lax.axis_index("s") * 2 + lax.axis_index("c")
    base = wid * _B_PER_W
    pltpu.sync_copy(idx_hbm.at[pl.ds(base, _B_PER_W)], idx_v)
    pltpu.async_copy(table_hbm.at[idx_v], rows_v, sem).wait()
    pltpu.sync_copy(rows_v, out_hbm.at[pl.ds(base, _B_PER_W)])


def _sc_gather(table, idx_padded):
    k = pl.kernel(
        _sc_gather_body,
        out_type=jax.ShapeDtypeStruct((_BP, E), jnp.float32),
        mesh=plsc.VectorSubcoreMesh(core_axis_name="c", subcore_axis_name="s"),
        scratch_types=[
            pltpu.VMEM((_B_PER_W,), jnp.int32),
            pltpu.VMEM((_B_PER_W, E), jnp.float32),
            pltpu.SemaphoreType.DMA,
        ],
    )
    return k(table, idx_padded)


TV = 2048


def _dense_body(x_ref, w1_ref, b1_ref, w2_ref, b2_ref, out_ref, h_ref):
    @pl.when(pl.program_id(0) == 0)
    def _():
        h = jnp.dot(x_ref[...], w1_ref[...],
                    preferred_element_type=jnp.float32,
                    precision=lax.Precision.HIGHEST)
        h_ref[...] = (h + b1_ref[...]).astype(jnp.bfloat16)

    acc = jnp.dot(h_ref[...], w2_ref[...].astype(jnp.bfloat16),
                  preferred_element_type=jnp.float32)
    acc = acc + b2_ref[...]
    out_ref[...] = acc.reshape(B, TP, TV)[:, :T, :]


def _dense(x, W1, b1, W2, b2):
    nt = pl.cdiv(V, TV)
    return pl.pallas_call(
        _dense_body,
        grid=(nt,),
        in_specs=[
            pl.BlockSpec((MP, E), lambda j: (0, 0)),
            pl.BlockSpec((E, H), lambda j: (0, 0)),
            pl.BlockSpec((1, H), lambda j: (0, 0)),
            pl.BlockSpec((H, TV), lambda j: (0, j)),
            pl.BlockSpec((1, TV), lambda j: (0, j)),
        ],
        out_specs=pl.BlockSpec((B, T, TV), lambda j: (0, 0, j)),
        out_shape=jax.ShapeDtypeStruct((B, T, V), jnp.float32),
        scratch_shapes=[pltpu.VMEM((MP, H), jnp.bfloat16)],
        compiler_params=pltpu.CompilerParams(
            dimension_semantics=("arbitrary",),
        ),
    )(x, W1, b1, W2, b2)


def kernel(features, captions, table, W1, b1, W2, b2):
    idx = captions.reshape(-1)
    idx_padded = jnp.concatenate(
        [idx, jnp.zeros((_BP - B * L,), jnp.int32)])
    emb = _sc_gather(table, idx_padded)[: B * L].reshape(B, L, E)
    pad = jnp.zeros((B, TP - T, E), jnp.float32)
    x = jnp.concatenate(
        [features[:, None, :], emb, pad], axis=1).reshape(MP, E)
    return _dense(x, W1, b1.reshape(1, H), W2, b2.reshape(1, V))
